# baseline (speedup 1.0000x reference)
import functools

import jax
import jax.numpy as jnp
from jax import lax
from jax.experimental import pallas as pl
from jax.experimental.pallas import tpu as pltpu
from jax.experimental.pallas import tpu_sc as plsc

N_NODES = 10000
N_EDGES = 160000
NPAD = 10240
EPAD = 163840
STRIPE = NPAD // 16
KB = 32
SLOTS = 8
NB_A = EPAD // (16 * KB)
NB_B = EPAD // (32 * KB)
FC = 128
MB = 1024

_mesh = functools.partial(
    plsc.VectorSubcoreMesh,
    core_axis_name="c", subcore_axis_name="s", num_cores=2, num_subcores=16)


NBH = NB_A // 8


def _agg_batches(h_hbm, src_v, dst_v, rows, acc, sem, nb):
    slot = [rows.at[pl.ds(j * KB, KB)] for j in range(SLOTS)]
    gsem = sem[:SLOTS]
    ssem = sem[SLOTS:]
    for j in range(SLOTS):
        pltpu.async_copy(h_hbm.at[src_v.at[j]], slot[j], gsem[j])

    def ring(r, carry):
        b0 = SLOTS * r
        for j in range(SLOTS):
            b = b0 + j
            bp = b - 1
            jp = (j - 1) % SLOTS

            @pl.when(bp >= 0)
            def _():
                pltpu.make_async_copy(
                    slot[jp], acc.at[dst_v.at[jp]], ssem[jp]).wait()

            @pl.when((bp >= 0) & (bp + SLOTS < nb))
            def _():
                pltpu.async_copy(h_hbm.at[src_v.at[bp + SLOTS]], slot[jp],
                                 gsem[jp])

            pltpu.make_async_copy(h_hbm.at[src_v.at[b]], slot[j],
                                  gsem[j]).wait()
            pltpu.async_copy(slot[j], acc.at[dst_v.at[b]], ssem[j],
                             add=True)
        return carry

    lax.fori_loop(0, nb // SLOTS, ring, 0)
    last = SLOTS - 1
    pltpu.make_async_copy(slot[last], acc.at[dst_v.at[nb - 1]],
                          ssem[last]).wait()


def _make_agg_colsplit(nc):
    npc = nc // 2

    @functools.partial(
        pl.kernel, mesh=_mesh(),
        out_type=jax.ShapeDtypeStruct((nc, NPAD, FC), jnp.float32),
        scratch_types=[
            pltpu.VMEM((NBH, KB), jnp.int32),
            pltpu.VMEM((NBH, KB), jnp.int32),
            pltpu.VMEM((SLOTS * KB, FC), jnp.float32),
            pltpu.VMEM_SHARED((NPAD, FC), jnp.float32),
            tuple(pltpu.SemaphoreType.DMA for _ in range(2 * SLOTS)),
        ])
    def agg(h_hbm, src_hbm, dst_hbm, z_hbm, out_hbm,
            src_v, dst_v, rows, acc, sem):
        c = lax.axis_index("c")
        s = lax.axis_index("s")
        for cc in range(npc):
            chunk = c * npc + cc
            pltpu.sync_copy(z_hbm, acc.at[pl.ds(s * STRIPE, STRIPE)])
            plsc.subcore_barrier()
            for hh in range(NB_A // NBH):
                pltpu.sync_copy(src_hbm.at[chunk, s, pl.ds(hh * NBH, NBH)],
                                src_v)
                pltpu.sync_copy(dst_hbm.at[s, pl.ds(hh * NBH, NBH)], dst_v)
                _agg_batches(h_hbm, src_v, dst_v, rows, acc, sem, NBH)
            plsc.subcore_barrier()
            pltpu.sync_copy(acc.at[pl.ds(s * STRIPE, STRIPE)],
                            out_hbm.at[chunk, pl.ds(s * STRIPE, STRIPE)])

    return agg


def _make_agg_edgesplit():
    @functools.partial(
        pl.kernel, mesh=_mesh(),
        out_type=jax.ShapeDtypeStruct((2, NPAD, FC), jnp.float32),
        scratch_types=[
            pltpu.VMEM((NBH, KB), jnp.int32),
            pltpu.VMEM((NBH, KB), jnp.int32),
            pltpu.VMEM((SLOTS * KB, FC), jnp.float32),
            pltpu.VMEM_SHARED((NPAD, FC), jnp.float32),
            tuple(pltpu.SemaphoreType.DMA for _ in range(2 * SLOTS)),
        ])
    def agg(h_hbm, src_hbm, dst_hbm, z_hbm, out_hbm,
            src_v, dst_v, rows, acc, sem):
        c = lax.axis_index("c")
        s = lax.axis_index("s")
        wid = c * 16 + s
        pltpu.sync_copy(z_hbm, acc.at[pl.ds(s * STRIPE, STRIPE)])
        plsc.subcore_barrier()
        for hh in range(NB_B // NBH):
            pltpu.sync_copy(src_hbm.at[wid, pl.ds(hh * NBH, NBH)], src_v)
            pltpu.sync_copy(dst_hbm.at[wid, pl.ds(hh * NBH, NBH)], dst_v)
            _agg_batches(h_hbm, src_v, dst_v, rows, acc, sem, NBH)
        plsc.subcore_barrier()
        pltpu.sync_copy(acc.at[pl.ds(s * STRIPE, STRIPE)],
                        out_hbm.at[c, pl.ds(s * STRIPE, STRIPE)])

    return agg


def _mm_chunked(a, w, relu):
    kc = a.shape[0]
    n_out = w.shape[1]
    nco = n_out // FC

    def body(a_ref, w_ref, o_ref):
        av = jnp.concatenate([a_ref[k] for k in range(kc)], axis=1)
        acc = jnp.dot(av.astype(jnp.bfloat16), w_ref[...].astype(jnp.bfloat16),
                      preferred_element_type=jnp.float32)
        if relu:
            acc = jnp.maximum(acc, 0.0)
        for n in range(nco):
            o_ref[n] = acc[:, n * FC:(n + 1) * FC]

    return pl.pallas_call(
        body,
        grid=(NPAD // MB,),
        in_specs=[
            pl.BlockSpec((kc, MB, FC), lambda m: (0, m, 0)),
            pl.BlockSpec((kc * FC, n_out), lambda m: (0, 0)),
        ],
        out_specs=pl.BlockSpec((nco, MB, FC), lambda m: (0, m, 0)),
        out_shape=jax.ShapeDtypeStruct((nco, NPAD, FC), jnp.float32),
    )(a, w)


def _mm_out(a, w):
    kc = a.shape[0]

    def body(a_ref, w_ref, o_ref):
        av = jnp.concatenate([a_ref[k] for k in range(kc)], axis=1)
        o_ref[...] = jnp.dot(av.astype(jnp.bfloat16),
                             w_ref[...].astype(jnp.bfloat16),
                             preferred_element_type=jnp.float32)

    return pl.pallas_call(
        body,
        grid=(NPAD // MB,),
        in_specs=[
            pl.BlockSpec((kc, MB, FC), lambda m: (0, m, 0)),
            pl.BlockSpec((kc * FC, FC), lambda m: (0, 0)),
        ],
        out_specs=pl.BlockSpec((MB, FC), lambda m: (m, 0)),
        out_shape=jax.ShapeDtypeStruct((NPAD, FC), jnp.float32),
    )(a, w)


def _logsoftmax_sum(parts):
    def body(p_ref, o_ref):
        x = p_ref[0, :, :64] + p_ref[1, :, :64]
        m = jnp.max(x, axis=1, keepdims=True)
        e = jnp.exp(x - m)
        lse = jnp.log(jnp.sum(e, axis=1, keepdims=True))
        o_ref[...] = x - m - lse

    return pl.pallas_call(
        body,
        grid=(NPAD // MB,),
        in_specs=[pl.BlockSpec((2, MB, FC), lambda m: (0, m, 0))],
        out_specs=pl.BlockSpec((MB, 64), lambda m: (m, 0)),
        out_shape=jax.ShapeDtypeStruct((NPAD, 64), jnp.float32),
    )(parts)


def kernel(x, edge_index, W_in, W_hid, W_out):
    src = edge_index[0]
    dst = edge_index[1]
    pad_e = EPAD - N_EDGES
    src_p = jnp.concatenate([src, jnp.zeros((pad_e,), jnp.int32)])
    dst_p = jnp.concatenate([dst, jnp.full((pad_e,), NPAD - 1, jnp.int32)])

    src_a = src_p.reshape(16, NB_A, KB)
    dst_a = dst_p.reshape(16, NB_A, KB)

    def src_chunked(nc):
        off = (jnp.arange(nc, dtype=jnp.int32) * NPAD)[:, None, None, None]
        return src_a[None] + off

    src_b = src_p.reshape(32, NB_B, KB)
    dst_b = dst_p.reshape(32, NB_B, KB)

    z = jnp.zeros((STRIPE, FC), jnp.float32)

    x_pad = jnp.pad(x, ((0, NPAD - N_NODES), (0, 0)))
    x_ch = x_pad.reshape(NPAD, 2, FC).transpose(1, 0, 2)

    w_out_p = jnp.pad(W_out, ((0, 0), (0, FC - 64)))

    agg2 = _make_agg_colsplit(2)
    agg4 = _make_agg_colsplit(4)
    agg_b = _make_agg_edgesplit()

    a1 = agg2(x_ch.reshape(2 * NPAD, FC), src_chunked(2), dst_a, z)
    h1 = _mm_chunked(a1, W_in, relu=True)

    a2 = agg4(h1.reshape(4 * NPAD, FC), src_chunked(4), dst_a, z)
    h2 = _mm_chunked(a2, W_hid, relu=True)

    h3 = _mm_out(h2, w_out_p)
    parts = agg_b(h3, src_b, dst_b, z)
    out = _logsoftmax_sum(parts)
    return out[:N_NODES]


# speedup vs baseline: 1.1497x; 1.1497x over previous
import functools

import jax
import jax.numpy as jnp
from jax import lax
from jax.experimental import pallas as pl
from jax.experimental.pallas import tpu as pltpu
from jax.experimental.pallas import tpu_sc as plsc

N_NODES = 10000
N_EDGES = 160000
NPAD = 10240
EPAD = 163840
STRIPE = NPAD // 16
KB = 64
SLOTS = 4
NB_A = EPAD // (16 * KB)
NB_B = EPAD // (32 * KB)
FC = 128
MB = 1024

_mesh = functools.partial(
    plsc.VectorSubcoreMesh,
    core_axis_name="c", subcore_axis_name="s", num_cores=2, num_subcores=16)


NBH = NB_A // 4


def _agg_batches(h_hbm, src_v, dst_v, rows, acc, sem, nb):
    slot = [rows.at[pl.ds(j * KB, KB)] for j in range(SLOTS)]
    gsem = sem[:SLOTS]
    ssem = sem[SLOTS:]
    for j in range(SLOTS):
        pltpu.async_copy(h_hbm.at[src_v.at[j]], slot[j], gsem[j])

    def ring(r, carry):
        b0 = SLOTS * r
        for j in range(SLOTS):
            b = b0 + j
            bp = b - 1
            jp = (j - 1) % SLOTS

            @pl.when(bp >= 0)
            def _():
                pltpu.make_async_copy(
                    slot[jp], acc.at[dst_v.at[jp]], ssem[jp]).wait()

            @pl.when((bp >= 0) & (bp + SLOTS < nb))
            def _():
                pltpu.async_copy(h_hbm.at[src_v.at[bp + SLOTS]], slot[jp],
                                 gsem[jp])

            pltpu.make_async_copy(h_hbm.at[src_v.at[b]], slot[j],
                                  gsem[j]).wait()
            pltpu.async_copy(slot[j], acc.at[dst_v.at[b]], ssem[j],
                             add=True)
        return carry

    lax.fori_loop(0, nb // SLOTS, ring, 0)
    last = SLOTS - 1
    pltpu.make_async_copy(slot[last], acc.at[dst_v.at[nb - 1]],
                          ssem[last]).wait()


def _make_agg_colsplit(nc):
    npc = nc // 2

    @functools.partial(
        pl.kernel, mesh=_mesh(),
        out_type=jax.ShapeDtypeStruct((nc, NPAD, FC), jnp.float32),
        scratch_types=[
            pltpu.VMEM((NBH, KB), jnp.int32),
            pltpu.VMEM((NBH, KB), jnp.int32),
            pltpu.VMEM((SLOTS * KB, FC), jnp.float32),
            pltpu.VMEM_SHARED((NPAD, FC), jnp.float32),
            tuple(pltpu.SemaphoreType.DMA for _ in range(2 * SLOTS)),
        ])
    def agg(h_hbm, src_hbm, dst_hbm, z_hbm, out_hbm,
            src_v, dst_v, rows, acc, sem):
        c = lax.axis_index("c")
        s = lax.axis_index("s")
        for cc in range(npc):
            chunk = c * npc + cc
            pltpu.sync_copy(z_hbm, acc.at[pl.ds(s * STRIPE, STRIPE)])
            plsc.subcore_barrier()
            for hh in range(NB_A // NBH):
                pltpu.sync_copy(src_hbm.at[chunk, s, pl.ds(hh * NBH, NBH)],
                                src_v)
                pltpu.sync_copy(dst_hbm.at[s, pl.ds(hh * NBH, NBH)], dst_v)
                _agg_batches(h_hbm, src_v, dst_v, rows, acc, sem, NBH)
            plsc.subcore_barrier()
            pltpu.sync_copy(acc.at[pl.ds(s * STRIPE, STRIPE)],
                            out_hbm.at[chunk, pl.ds(s * STRIPE, STRIPE)])

    return agg


def _make_agg_edgesplit():
    @functools.partial(
        pl.kernel, mesh=_mesh(),
        out_type=jax.ShapeDtypeStruct((2, NPAD, FC), jnp.float32),
        scratch_types=[
            pltpu.VMEM((NBH, KB), jnp.int32),
            pltpu.VMEM((NBH, KB), jnp.int32),
            pltpu.VMEM((SLOTS * KB, FC), jnp.float32),
            pltpu.VMEM_SHARED((NPAD, FC), jnp.float32),
            tuple(pltpu.SemaphoreType.DMA for _ in range(2 * SLOTS)),
        ])
    def agg(h_hbm, src_hbm, dst_hbm, z_hbm, out_hbm,
            src_v, dst_v, rows, acc, sem):
        c = lax.axis_index("c")
        s = lax.axis_index("s")
        wid = c * 16 + s
        pltpu.sync_copy(z_hbm, acc.at[pl.ds(s * STRIPE, STRIPE)])
        plsc.subcore_barrier()
        for hh in range(NB_B // NBH):
            pltpu.sync_copy(src_hbm.at[wid, pl.ds(hh * NBH, NBH)], src_v)
            pltpu.sync_copy(dst_hbm.at[wid, pl.ds(hh * NBH, NBH)], dst_v)
            _agg_batches(h_hbm, src_v, dst_v, rows, acc, sem, NBH)
        plsc.subcore_barrier()
        pltpu.sync_copy(acc.at[pl.ds(s * STRIPE, STRIPE)],
                        out_hbm.at[c, pl.ds(s * STRIPE, STRIPE)])

    return agg


def _mm_chunked(a, w, relu):
    kc = a.shape[0]
    n_out = w.shape[1]
    nco = n_out // FC

    def body(a_ref, w_ref, o_ref):
        av = jnp.concatenate([a_ref[k] for k in range(kc)], axis=1)
        acc = jnp.dot(av.astype(jnp.bfloat16), w_ref[...].astype(jnp.bfloat16),
                      preferred_element_type=jnp.float32)
        if relu:
            acc = jnp.maximum(acc, 0.0)
        for n in range(nco):
            o_ref[n] = acc[:, n * FC:(n + 1) * FC]

    return pl.pallas_call(
        body,
        grid=(NPAD // MB,),
        in_specs=[
            pl.BlockSpec((kc, MB, FC), lambda m: (0, m, 0)),
            pl.BlockSpec((kc * FC, n_out), lambda m: (0, 0)),
        ],
        out_specs=pl.BlockSpec((nco, MB, FC), lambda m: (0, m, 0)),
        out_shape=jax.ShapeDtypeStruct((nco, NPAD, FC), jnp.float32),
    )(a, w)


def _mm_fused_out(a, w1, w2):
    kc = a.shape[0]

    def body(a_ref, w1_ref, w2_ref, o_ref):
        av = jnp.concatenate([a_ref[k] for k in range(kc)], axis=1)
        h = jnp.dot(av.astype(jnp.bfloat16), w1_ref[...].astype(jnp.bfloat16),
                    preferred_element_type=jnp.float32)
        h = jnp.maximum(h, 0.0)
        o_ref[...] = jnp.dot(h.astype(jnp.bfloat16),
                             w2_ref[...].astype(jnp.bfloat16),
                             preferred_element_type=jnp.float32)

    return pl.pallas_call(
        body,
        grid=(NPAD // MB,),
        in_specs=[
            pl.BlockSpec((kc, MB, FC), lambda m: (0, m, 0)),
            pl.BlockSpec((kc * FC, 512), lambda m: (0, 0)),
            pl.BlockSpec((512, FC), lambda m: (0, 0)),
        ],
        out_specs=pl.BlockSpec((MB, FC), lambda m: (m, 0)),
        out_shape=jax.ShapeDtypeStruct((NPAD, FC), jnp.float32),
    )(a, w1, w2)


def _logsoftmax_sum(parts):
    def body(p_ref, o_ref):
        x = p_ref[0, :, :64] + p_ref[1, :, :64]
        m = jnp.max(x, axis=1, keepdims=True)
        e = jnp.exp(x - m)
        lse = jnp.log(jnp.sum(e, axis=1, keepdims=True))
        o_ref[...] = x - m - lse

    return pl.pallas_call(
        body,
        grid=(NPAD // MB,),
        in_specs=[pl.BlockSpec((2, MB, FC), lambda m: (0, m, 0))],
        out_specs=pl.BlockSpec((MB, 64), lambda m: (m, 0)),
        out_shape=jax.ShapeDtypeStruct((NPAD, 64), jnp.float32),
    )(parts)


def kernel(x, edge_index, W_in, W_hid, W_out):
    src = edge_index[0]
    dst = edge_index[1]
    pad_e = EPAD - N_EDGES
    src_p = jnp.concatenate([src, jnp.zeros((pad_e,), jnp.int32)])
    dst_p = jnp.concatenate([dst, jnp.full((pad_e,), NPAD - 1, jnp.int32)])

    src_a = src_p.reshape(16, NB_A, KB)
    dst_a = dst_p.reshape(16, NB_A, KB)

    def src_chunked(nc):
        off = (jnp.arange(nc, dtype=jnp.int32) * NPAD)[:, None, None, None]
        return src_a[None] + off

    src_b = src_p.reshape(32, NB_B, KB)
    dst_b = dst_p.reshape(32, NB_B, KB)

    z = jnp.zeros((STRIPE, FC), jnp.float32)

    x_pad = jnp.pad(x, ((0, NPAD - N_NODES), (0, 0)))
    x_ch = x_pad.reshape(NPAD, 2, FC).transpose(1, 0, 2)

    w_out_p = jnp.pad(W_out, ((0, 0), (0, FC - 64)))

    agg2 = _make_agg_colsplit(2)
    agg4 = _make_agg_colsplit(4)
    agg_b = _make_agg_edgesplit()

    a1 = agg2(x_ch.reshape(2 * NPAD, FC), src_chunked(2), dst_a, z)
    h1 = _mm_chunked(a1, W_in, relu=True)

    a2 = agg4(h1.reshape(4 * NPAD, FC), src_chunked(4), dst_a, z)
    h3 = _mm_fused_out(a2, W_hid, w_out_p)
    parts = agg_b(h3, src_b, dst_b, z)
    out = _logsoftmax_sum(parts)
    return out[:N_NODES]
